# Initial kernel scaffold; baseline (speedup 1.0000x reference)
import jax
import jax.numpy as jnp
from jax.experimental import pallas as pl


def kernel(x, table):
    raise NotImplementedError("write your pallas kernel here")


# speedup vs baseline: 1.8376x; 1.8376x over previous
import functools

import jax
import jax.numpy as jnp
from jax import lax
from jax.experimental import pallas as pl
from jax.experimental.pallas import tpu as pltpu
from jax.experimental.pallas import tpu_sc as plsc

NC = 2
NS = 16
NW = NC * NS
CH = 128


@functools.partial(jax.jit, static_argnums=(2, 3))
def _sc_gather(table, idx, B, D):
    nch = idx.shape[1]
    bpw = nch * CH
    mesh = plsc.VectorSubcoreMesh(core_axis_name="c", subcore_axis_name="s")

    @functools.partial(
        pl.kernel,
        mesh=mesh,
        out_type=jax.ShapeDtypeStruct((B, D), jnp.float32),
        compiler_params=pltpu.CompilerParams(use_tc_tiling_on_sc=False),
        scratch_types=[
            pltpu.VMEM((nch, CH), jnp.int32),
            pltpu.VMEM((CH, D), jnp.float32),
            pltpu.VMEM((CH, D), jnp.float32),
            pltpu.SemaphoreType.DMA,
            pltpu.SemaphoreType.DMA,
        ],
    )
    def k(table_hbm, idx_hbm, out_hbm, idx_v, buf0, buf1, sem0, sem1):
        c = lax.axis_index("c")
        s = lax.axis_index("s")
        wid = s * NC + c
        base = wid * bpw
        pltpu.sync_copy(idx_hbm.at[wid], idx_v)
        pltpu.make_async_copy(table_hbm.at[idx_v.at[0]], buf0, sem0).start()

        nhalf = nch // 2

        def body(jj, _):
            j0 = 2 * jj
            j1 = j0 + 1
            pltpu.make_async_copy(table_hbm.at[idx_v.at[j1]], buf1, sem1).start()
            pltpu.make_async_copy(table_hbm.at[idx_v.at[j0]], buf0, sem0).wait()
            pltpu.sync_copy(buf0, out_hbm.at[pl.ds(base + j0 * CH, CH)])

            @pl.when(jj + 1 < nhalf)
            def _():
                pltpu.make_async_copy(
                    table_hbm.at[idx_v.at[j0 + 2]], buf0, sem0).start()

            pltpu.make_async_copy(table_hbm.at[idx_v.at[j1]], buf1, sem1).wait()
            pltpu.sync_copy(buf1, out_hbm.at[pl.ds(base + j1 * CH, CH)])
            return 0

        lax.fori_loop(0, nhalf, body, 0)

    return k(table, idx)


def kernel(x, table):
    B = x.shape[0] * x.shape[1]
    D = table.shape[1]
    idx = x.reshape(NW, B // (NW * CH), CH).astype(jnp.int32)
    out = _sc_gather(table, idx, B, D)
    return out.reshape(x.shape[0], x.shape[1], D)


# speedup vs baseline: 1.8625x; 1.0135x over previous
import functools

import jax
import jax.numpy as jnp
from jax import lax
from jax.experimental import pallas as pl
from jax.experimental.pallas import tpu as pltpu
from jax.experimental.pallas import tpu_sc as plsc

NC = 2
NS = 16
NW = NC * NS
CH = 128
R = 4
AH = 2


@functools.partial(jax.jit, static_argnums=(2, 3))
def _sc_gather(table, idx, B, D):
    nch = idx.shape[1]
    bpw = nch * CH
    mesh = plsc.VectorSubcoreMesh(core_axis_name="c", subcore_axis_name="s")

    @functools.partial(
        pl.kernel,
        mesh=mesh,
        out_type=jax.ShapeDtypeStruct((B, D), jnp.float32),
        compiler_params=pltpu.CompilerParams(use_tc_tiling_on_sc=False),
        scratch_types=(
            [pltpu.VMEM((nch, CH), jnp.int32)]
            + [pltpu.VMEM((CH, D), jnp.float32) for _ in range(R)]
            + [pltpu.SemaphoreType.DMA for _ in range(2 * R)]
        ),
    )
    def k(table_hbm, idx_hbm, out_hbm, idx_v, *rest):
        bufs = rest[:R]
        gsems = rest[R:2 * R]
        osems = rest[2 * R:3 * R]
        c = lax.axis_index("c")
        s = lax.axis_index("s")
        wid = s * NC + c
        base = wid * bpw
        pltpu.sync_copy(idx_hbm.at[wid], idx_v)
        for b in range(AH):
            pltpu.make_async_copy(
                table_hbm.at[idx_v.at[b]], bufs[b], gsems[b]).start()

        nq = nch // R

        def body(q, _):
            j0 = q * R
            for b in range(R):
                j = j0 + b
                bn = (b + AH) % R
                pltpu.make_async_copy(
                    table_hbm.at[idx_v.at[j]], bufs[b], gsems[b]).wait()
                pltpu.make_async_copy(
                    bufs[b], out_hbm.at[pl.ds(base + j * CH, CH)],
                    osems[b]).start()

                @pl.when(j + AH < nch)
                def _():
                    @pl.when(j >= AH)
                    def _():
                        pltpu.make_async_copy(
                            bufs[bn],
                            out_hbm.at[pl.ds(base + (j - AH) * CH, CH)],
                            osems[bn]).wait()
                    pltpu.make_async_copy(
                        table_hbm.at[idx_v.at[j + AH]], bufs[bn],
                        gsems[bn]).start()
            return 0

        lax.fori_loop(0, nq, body, 0)
        for b in range(R):
            pltpu.make_async_copy(
                bufs[b], out_hbm.at[pl.ds(base + (nch - R + b) * CH, CH)],
                osems[b]).wait()

    return k(table, idx)


def kernel(x, table):
    B = x.shape[0] * x.shape[1]
    D = table.shape[1]
    idx = x.reshape(NW, B // (NW * CH), CH).astype(jnp.int32)
    out = _sc_gather(table, idx, B, D)
    return out.reshape(x.shape[0], x.shape[1], D)
